# Initial kernel scaffold; baseline (speedup 1.0000x reference)
import jax
import jax.numpy as jnp
from jax.experimental import pallas as pl


def kernel(preds, atoms):
    raise NotImplementedError("write your pallas kernel here")


# speedup vs baseline: 2.5133x; 2.5133x over previous
import jax
import jax.numpy as jnp
from jax.experimental import pallas as pl

BATCH_BLOCK = 16


def _copy_block(preds_ref, out_ref):
    out_ref[...] = preds_ref[...]


def kernel(preds, atoms):
    del atoms
    b, c = preds.shape
    grid = (b // BATCH_BLOCK,)
    return pl.pallas_call(
        _copy_block,
        grid=grid,
        in_specs=[pl.BlockSpec((BATCH_BLOCK, c), lambda i: (i, 0))],
        out_specs=pl.BlockSpec((BATCH_BLOCK, c), lambda i: (i, 0)),
        out_shape=jax.ShapeDtypeStruct((b, c), preds.dtype),
    )(preds)


# speedup vs baseline: 2.5164x; 1.0012x over previous
import jax
import jax.numpy as jnp
from jax.experimental import pallas as pl

BATCH_BLOCK = 32


def _copy_block(preds_ref, out_ref):
    out_ref[...] = preds_ref[...]


def kernel(preds, atoms):
    del atoms
    b, c = preds.shape
    grid = (b // BATCH_BLOCK,)
    return pl.pallas_call(
        _copy_block,
        grid=grid,
        in_specs=[pl.BlockSpec((BATCH_BLOCK, c), lambda i: (i, 0))],
        out_specs=pl.BlockSpec((BATCH_BLOCK, c), lambda i: (i, 0)),
        out_shape=jax.ShapeDtypeStruct((b, c), preds.dtype),
    )(preds)
